# baseline (speedup 1.0000x reference)
import functools

import jax
import jax.numpy as jnp
from jax import lax
from jax.experimental import pallas as pl
from jax.experimental.pallas import tpu as pltpu
from jax.experimental.pallas import tpu_sc as plsc

_B = 2
_T = 4096
_H = 4096
_K = 8
_SLOTS = 64
_TB = 256
_NT = _T // _TB

_NC = 2
_NS = 16
_NW = _NC * _NS


def _gate_body(w_ref, b_ref, x_ref, scores_ref, idx_ref, acc_ref):
    bi = pl.program_id(0)
    ti = pl.program_id(1)
    x = x_ref[0]
    w = w_ref[...]
    logits = lax.dot_general(
        w, x, (((1,), (1,)), ((), ())), preferred_element_type=jnp.float32
    )
    logits = logits + b_ref[0, 0]
    scores_ref[...] = jax.nn.sigmoid(logits)[None, None]
    acc_ref[pl.ds(ti, 1), :] = logits

    @pl.when(ti == _NT - 1)
    def _():
        vals = acc_ref[...]
        rows = lax.broadcasted_iota(jnp.int32, (_NT, _TB), 0)
        cols = lax.broadcasted_iota(jnp.int32, (_NT, _TB), 1)
        gpos = rows * _TB + cols
        big = jnp.int32(_T)
        neg = jnp.float32(-jnp.inf)
        for j in range(_K):
            m = jnp.max(vals)
            ij = jnp.min(jnp.where(vals == m, gpos, big))
            idx_ref[bi, j] = bi * _T + ij
            vals = jnp.where(gpos == ij, neg, vals)


def _gate(enc, w1h, b2d):
    return pl.pallas_call(
        _gate_body,
        grid=(_B, _NT),
        in_specs=[
            pl.BlockSpec((1, _H), lambda b, t: (0, 0)),
            pl.BlockSpec(memory_space=pltpu.SMEM),
            pl.BlockSpec((1, _TB, _H), lambda b, t: (b, t, 0)),
        ],
        out_specs=[
            pl.BlockSpec((1, 1, 1, _TB), lambda b, t: (b, t, 0, 0)),
            pl.BlockSpec(memory_space=pltpu.SMEM),
        ],
        out_shape=[
            jax.ShapeDtypeStruct((_B, _NT, 1, _TB), jnp.float32),
            jax.ShapeDtypeStruct((_B, _K), jnp.int32),
        ],
        scratch_shapes=[pltpu.VMEM((_NT, _TB), jnp.float32)],
    )(w1h, b2d, enc)


def _sc_write_memory(enc2d, gidx):
    mesh = plsc.VectorSubcoreMesh(core_axis_name="c", subcore_axis_name="s")
    nz = _B * (_SLOTS - _K)

    @functools.partial(
        pl.kernel,
        mesh=mesh,
        out_type=jax.ShapeDtypeStruct((_B * _SLOTS, _H), jnp.float32),
        scratch_types=[
            pltpu.VMEM((_B * _K,), jnp.int32),
            pltpu.VMEM((_B * _K,), jnp.int32),
            pltpu.VMEM((_B * _K, _H), jnp.float32),
            pltpu.VMEM((_H,), jnp.float32),
            pltpu.SemaphoreType.DMA,
        ],
    )
    def k(enc_hbm, gidx_hbm, out_hbm, idx_v, oidx_v, rows_v, zrow_v, sem):
        cid = lax.axis_index("c")
        sid = lax.axis_index("s")
        wid = sid * _NC + cid

        z16 = jnp.zeros((16,), jnp.float32)
        for i in range(0, _H, 16):
            zrow_v[pl.ds(i, 16)] = z16
        for r in range(4):
            n = wid + _NW * r

            @pl.when(n < nz)
            def _():
                row = _K + n + jnp.where(n >= _SLOTS - _K, _K, 0)
                pltpu.sync_copy(zrow_v, out_hbm.at[row])

        @pl.when(wid == 0)
        def _():
            pltpu.sync_copy(gidx_hbm, idx_v)
            pltpu.async_copy(enc_hbm.at[idx_v], rows_v, sem).wait()
            ii = lax.iota(jnp.int32, _B * _K)
            oidx_v[...] = ii + jnp.where(ii >= _K, _SLOTS - _K, 0)
            pltpu.async_copy(rows_v, out_hbm.at[oidx_v], sem).wait()

    return k(enc2d, gidx)


def kernel(enc_hidden, W, b):
    w1h = W.reshape(1, _H)
    b2d = b.reshape(1, 1)
    scores3d, gidx = _gate(enc_hidden, w1h, b2d)
    gate_scores = scores3d.reshape(_B, _T)
    enc2d = enc_hidden.reshape(_B * _T, _H)
    mem2d = _sc_write_memory(enc2d, gidx.reshape(_B * _K))
    memory = mem2d.reshape(_B, _SLOTS, _H)
    return (memory, gate_scores)


# speedup vs baseline: 1.1175x; 1.1175x over previous
import functools

import jax
import jax.numpy as jnp
from jax import lax
from jax.experimental import pallas as pl
from jax.experimental.pallas import tpu as pltpu
from jax.experimental.pallas import tpu_sc as plsc

_B = 2
_T = 4096
_H = 4096
_K = 8
_SLOTS = 64
_TB = 1024
_NT = _T // _TB

_NC = 2
_NS = 16
_NW = _NC * _NS


def _gate_body(w_ref, b_ref, x_ref, scores_ref, idx_ref, acc_ref):
    bi = pl.program_id(0)
    ti = pl.program_id(1)
    x = x_ref[0]
    w = w_ref[...]
    logits = lax.dot_general(
        w, x, (((1,), (1,)), ((), ())), preferred_element_type=jnp.float32
    )
    logits = logits + b_ref[0, 0]
    scores_ref[...] = jax.nn.sigmoid(logits)[None, None]
    acc_ref[pl.ds(ti, 1), :] = logits

    @pl.when(ti == _NT - 1)
    def _():
        vals = acc_ref[...]
        rows = lax.broadcasted_iota(jnp.int32, (_NT, _TB), 0)
        cols = lax.broadcasted_iota(jnp.int32, (_NT, _TB), 1)
        gpos = rows * _TB + cols
        big = jnp.int32(_T)
        neg = jnp.float32(-jnp.inf)
        for j in range(_K):
            m = jnp.max(vals)
            ij = jnp.min(jnp.where(vals == m, gpos, big))
            idx_ref[bi, j] = bi * _T + ij
            vals = jnp.where(gpos == ij, neg, vals)


def _gate(enc, w1h, b2d):
    return pl.pallas_call(
        _gate_body,
        grid=(_B, _NT),
        in_specs=[
            pl.BlockSpec((1, _H), lambda b, t: (0, 0)),
            pl.BlockSpec(memory_space=pltpu.SMEM),
            pl.BlockSpec((1, _TB, _H), lambda b, t: (b, t, 0)),
        ],
        out_specs=[
            pl.BlockSpec((1, 1, 1, _TB), lambda b, t: (b, t, 0, 0)),
            pl.BlockSpec(memory_space=pltpu.SMEM),
        ],
        out_shape=[
            jax.ShapeDtypeStruct((_B, _NT, 1, _TB), jnp.float32),
            jax.ShapeDtypeStruct((_B, _K), jnp.int32),
        ],
        scratch_shapes=[pltpu.VMEM((_NT, _TB), jnp.float32)],
    )(w1h, b2d, enc)


def _sc_write_memory(enc2d, gidx, oidx):
    mesh = plsc.VectorSubcoreMesh(core_axis_name="c", subcore_axis_name="s")

    @functools.partial(
        pl.kernel,
        mesh=mesh,
        out_type=jax.ShapeDtypeStruct((_B * _SLOTS, _H), jnp.float32),
        scratch_types=[
            pltpu.VMEM((_K,), jnp.int32),
            pltpu.VMEM((_K,), jnp.int32),
            pltpu.VMEM((_K, _H), jnp.float32),
            pltpu.VMEM((4, _H), jnp.float32),
            pltpu.SemaphoreType.DMA,
        ],
    )
    def k(enc_hbm, gidx_hbm, oidx_hbm, out_hbm, idx_v, oidx_v, rows_v, zbuf_v, sem):
        cid = lax.axis_index("c")
        sid = lax.axis_index("s")
        wid = sid * _NC + cid

        @pl.when(wid < 28)
        def _():
            z16 = jnp.zeros((16,), jnp.float32)
            for r in range(4):
                for i in range(0, _H, 16):
                    zbuf_v[r, pl.ds(i, 16)] = z16
            row = jnp.where(wid < 14, _K + 4 * wid, _SLOTS + _K + 4 * (wid - 14))
            pltpu.sync_copy(zbuf_v, out_hbm.at[pl.ds(row, 4)])

        for w in range(_B):

            @pl.when(wid == 28 + w)
            def _(w=w):
                pltpu.sync_copy(gidx_hbm.at[pl.ds(_K * w, _K)], idx_v)
                pltpu.sync_copy(oidx_hbm.at[pl.ds(_K * w, _K)], oidx_v)
                pltpu.async_copy(enc_hbm.at[idx_v], rows_v, sem).wait()
                pltpu.async_copy(rows_v, out_hbm.at[oidx_v], sem).wait()

    return k(enc2d, gidx, oidx)


def kernel(enc_hidden, W, b):
    w1h = W.reshape(1, _H)
    b2d = b.reshape(1, 1)
    scores3d, gidx = _gate(enc_hidden, w1h, b2d)
    gate_scores = scores3d.reshape(_B, _T)
    enc2d = enc_hidden.reshape(_B * _T, _H)
    oidx = (jnp.arange(_B * _K, dtype=jnp.int32)
            + jnp.where(jnp.arange(_B * _K) >= _K, _SLOTS - _K, 0))
    mem2d = _sc_write_memory(enc2d, gidx.reshape(_B * _K), oidx)
    memory = mem2d.reshape(_B, _SLOTS, _H)
    return (memory, gate_scores)


# speedup vs baseline: 1.1495x; 1.0286x over previous
import functools

import jax
import jax.numpy as jnp
import numpy as np
from jax import lax
from jax.experimental import pallas as pl
from jax.experimental.pallas import tpu as pltpu
from jax.experimental.pallas import tpu_sc as plsc

_B = 2
_T = 4096
_H = 4096
_K = 8
_SLOTS = 64
_TB = 1024
_NT = _T // _TB

_NC = 2
_NS = 16
_NW = _NC * _NS

_OIDX = np.arange(_B * _K, dtype=np.int32) + np.where(
    np.arange(_B * _K) >= _K, _SLOTS - _K, 0
).astype(np.int32)


def _gate_body(w_ref, b_ref, x_ref, scores_ref, idx_ref, acc_ref):
    bi = pl.program_id(0)
    ti = pl.program_id(1)
    x = x_ref[0]
    w = w_ref[...]
    logits = lax.dot_general(
        w, x, (((1,), (1,)), ((), ())), preferred_element_type=jnp.float32
    )
    logits = logits + b_ref[0, 0]
    scores_ref[...] = jax.nn.sigmoid(logits)[0]
    acc_ref[pl.ds(ti, 1), :] = logits

    @pl.when(ti == _NT - 1)
    def _():
        vals = acc_ref[...]
        rows = lax.broadcasted_iota(jnp.int32, (_NT, _TB), 0)
        cols = lax.broadcasted_iota(jnp.int32, (_NT, _TB), 1)
        gpos = rows * _TB + cols
        big = jnp.int32(_T)
        neg = jnp.float32(-jnp.inf)
        for j in range(_K):
            m = jnp.max(vals)
            ij = jnp.min(jnp.where(vals == m, gpos, big))
            idx_ref[bi * _K + j] = bi * _T + ij
            vals = jnp.where(gpos == ij, neg, vals)


def _gate(enc, w1h, b2d):
    return pl.pallas_call(
        _gate_body,
        grid=(_B, _NT),
        in_specs=[
            pl.BlockSpec((1, _H), lambda b, t: (0, 0)),
            pl.BlockSpec(memory_space=pltpu.SMEM),
            pl.BlockSpec((1, _TB, _H), lambda b, t: (b, t, 0)),
        ],
        out_specs=[
            pl.BlockSpec((_TB,), lambda b, t: (b * _NT + t,)),
            pl.BlockSpec(memory_space=pltpu.SMEM),
        ],
        out_shape=[
            jax.ShapeDtypeStruct((_B * _T,), jnp.float32),
            jax.ShapeDtypeStruct((_B * _K,), jnp.int32),
        ],
        scratch_shapes=[pltpu.VMEM((_NT, _TB), jnp.float32)],
    )(w1h, b2d, enc)


def _sc_write_memory(enc2d, gidx, oidx):
    mesh = plsc.VectorSubcoreMesh(core_axis_name="c", subcore_axis_name="s")

    @functools.partial(
        pl.kernel,
        mesh=mesh,
        out_type=jax.ShapeDtypeStruct((_B * _SLOTS, _H), jnp.float32),
        scratch_types=[
            pltpu.VMEM((_K,), jnp.int32),
            pltpu.VMEM((_K,), jnp.int32),
            pltpu.VMEM((_K, _H), jnp.float32),
            pltpu.VMEM((_H,), jnp.float32),
            pltpu.SemaphoreType.DMA,
        ],
    )
    def k(enc_hbm, gidx_hbm, oidx_hbm, out_hbm, idx_v, oidx_v, rows_v, zrow_v, sem):
        cid = lax.axis_index("c")
        sid = lax.axis_index("s")
        wid = sid * _NC + cid

        @pl.when(wid < 28)
        def _():
            z16 = jnp.zeros((16,), jnp.float32)
            for i in range(0, _H, 16):
                zrow_v[pl.ds(i, 16)] = z16
            row0 = jnp.where(wid < 14, _K + 4 * wid, _SLOTS + _K + 4 * (wid - 14))
            copies = [
                pltpu.async_copy(zrow_v, out_hbm.at[row0 + r], sem)
                for r in range(4)
            ]
            for c in copies:
                c.wait()

        for w in range(_B):

            @pl.when(wid == 28 + w)
            def _(w=w):
                c1 = pltpu.async_copy(gidx_hbm.at[pl.ds(_K * w, _K)], idx_v, sem)
                c2 = pltpu.async_copy(oidx_hbm.at[pl.ds(_K * w, _K)], oidx_v, sem)
                c1.wait()
                c2.wait()
                pltpu.async_copy(enc_hbm.at[idx_v], rows_v, sem).wait()
                pltpu.async_copy(rows_v, out_hbm.at[oidx_v], sem).wait()

    return k(enc2d, gidx, oidx)


def kernel(enc_hidden, W, b):
    w1h = W.reshape(1, _H)
    b2d = b.reshape(1, 1)
    scores_flat, gidx = _gate(enc_hidden, w1h, b2d)
    gate_scores = scores_flat.reshape(_B, _T)
    enc2d = enc_hidden.reshape(_B * _T, _H)
    mem2d = _sc_write_memory(enc2d, gidx, jnp.asarray(_OIDX))
    memory = mem2d.reshape(_B, _SLOTS, _H)
    return (memory, gate_scores)


# speedup vs baseline: 1.4654x; 1.2749x over previous
import functools

import jax
import jax.numpy as jnp
import numpy as np
from jax import lax
from jax.experimental import pallas as pl
from jax.experimental.pallas import tpu as pltpu
from jax.experimental.pallas import tpu_sc as plsc

_B = 2
_T = 4096
_H = 4096
_K = 8
_SLOTS = 64
_TB = 1024
_NT = _T // _TB

_NC = 2
_NS = 16
_NW = _NC * _NS

_OIDX = np.arange(_B * _K, dtype=np.int32) + np.where(
    np.arange(_B * _K) >= _K, _SLOTS - _K, 0
).astype(np.int32)


def _gate_body(w_ref, b_ref, x_ref, scores_ref, idx_ref, acc_ref):
    bi = pl.program_id(0)
    ti = pl.program_id(1)
    x = x_ref[0]
    w = w_ref[...]
    logits = lax.dot_general(
        w, x, (((1,), (1,)), ((), ())), preferred_element_type=jnp.float32
    )
    logits = logits + b_ref[0, 0]
    scores_ref[...] = jax.nn.sigmoid(logits)[0]
    acc_ref[pl.ds(ti, 1), :] = logits

    @pl.when(ti == _NT - 1)
    def _():
        vals = acc_ref[...]
        rows = lax.broadcasted_iota(jnp.int32, (_NT, _TB), 0)
        cols = lax.broadcasted_iota(jnp.int32, (_NT, _TB), 1)
        gpos = rows * _TB + cols
        big = jnp.int32(_T)
        neg = jnp.float32(-jnp.inf)
        for j in range(_K):
            m = jnp.max(vals)
            ij = jnp.min(jnp.where(vals == m, gpos, big))
            idx_ref[bi * _K + j] = bi * _T + ij
            vals = jnp.where(gpos == ij, neg, vals)


def _gate(enc, w1h, b2d):
    return pl.pallas_call(
        _gate_body,
        grid=(_B, _NT),
        in_specs=[
            pl.BlockSpec((1, _H), lambda b, t: (0, 0)),
            pl.BlockSpec(memory_space=pltpu.SMEM),
            pl.BlockSpec((1, _TB, _H), lambda b, t: (b, t, 0)),
        ],
        out_specs=[
            pl.BlockSpec((_TB,), lambda b, t: (b * _NT + t,)),
            pl.BlockSpec(memory_space=pltpu.SMEM),
        ],
        out_shape=[
            jax.ShapeDtypeStruct((_B * _T,), jnp.float32),
            jax.ShapeDtypeStruct((_B * _K,), jnp.int32),
        ],
        scratch_shapes=[pltpu.VMEM((_NT, _TB), jnp.float32)],
    )(w1h, b2d, enc)


def _sc_write_memory(enc2d, gidx, oidx):
    mesh = plsc.VectorSubcoreMesh(core_axis_name="c", subcore_axis_name="s")

    @functools.partial(
        pl.kernel,
        mesh=mesh,
        out_type=jax.ShapeDtypeStruct((_B * _SLOTS, _H), jnp.float32),
        scratch_types=[
            pltpu.VMEM((_K,), jnp.int32),
            pltpu.VMEM((_K,), jnp.int32),
            pltpu.VMEM((_K, _H), jnp.float32),
            pltpu.VMEM((_H,), jnp.float32),
            pltpu.SemaphoreType.DMA,
        ],
    )
    def k(enc_hbm, gidx_hbm, oidx_hbm, out_hbm, idx_v, oidx_v, rows_v, zrow_v, sem):
        cid = lax.axis_index("c")
        sid = lax.axis_index("s")
        wid = sid * _NC + cid

        @pl.when(wid < 28)
        def _():
            z16 = jnp.zeros((16,), jnp.float32)
            for i in range(0, _H, 16):
                zrow_v[pl.ds(i, 16)] = z16
            row0 = jnp.where(wid < 14, _K + 4 * wid, _SLOTS + _K + 4 * (wid - 14))
            copies = [
                pltpu.async_copy(zrow_v, out_hbm.at[row0 + r], sem)
                for r in range(4)
            ]
            for c in copies:
                c.wait()

        for w in range(_B):

            @pl.when(wid == 28 + w)
            def _(w=w):
                c1 = pltpu.async_copy(gidx_hbm.at[pl.ds(_K * w, _K)], idx_v, sem)
                c2 = pltpu.async_copy(oidx_hbm.at[pl.ds(_K * w, _K)], oidx_v, sem)
                c1.wait()
                c2.wait()
                pltpu.async_copy(enc_hbm.at[idx_v], rows_v, sem).wait()
                pltpu.async_copy(rows_v, out_hbm.at[oidx_v], sem).wait()

    return k(enc2d, gidx, oidx)


def kernel(enc_hidden, W, b):
    w1h = W.reshape(1, _H)
    b2d = b.reshape(1, 1)
    scores_flat, gidx = _gate(enc_hidden, w1h, b2d)
    gate_scores = scores_flat.reshape(_B, _T)
    enc2d = enc_hidden.reshape(_B * _T, _H)
    gathered = jnp.take(enc2d, gidx, axis=0).reshape(_B, _K, _H)
    memory = jnp.zeros((_B, _SLOTS, _H), jnp.float32).at[:, :_K, :].set(gathered)
    return (memory, gate_scores)
